# Initial kernel scaffold; baseline (speedup 1.0000x reference)
import jax
import jax.numpy as jnp
from jax.experimental import pallas as pl


def kernel(trees, indexes, mask_padding, W_emb, b_emb, w_attn, W_mlp, b_mlp, W_cost, b_cost, W_card, b_card):
    raise NotImplementedError("write your pallas kernel here")


# speedup vs baseline: 7.0776x; 7.0776x over previous
import jax
import jax.numpy as jnp
from jax.experimental import pallas as pl
from jax.experimental.pallas import tpu as pltpu

_B = 512
_N = 128
_F = 128
_H = 256
_MO = 64
_P = 8


def _fused_kernel(trees_ref, idx_ref, valid_ref, Wemb_ref, bemb_ref, wattn_ref,
                  Wmlp_ref, bmlp_ref, Wc_ref, bc_ref, Wk_ref, bk_ref,
                  cost_ref, card_ref, mlp_ref):
    f32 = jnp.float32
    Wemb = Wemb_ref[...]
    bemb = bemb_ref[...]
    wattn = wattn_ref[...]
    iota_m = jax.lax.broadcasted_iota(jnp.int32, (_N, _N), 0)
    combs = []
    for p in range(_P):
        A = trees_ref[p]
        proj = jax.lax.dot_general(A, Wemb, (((0,), (0,)), ((), ())),
                                   preferred_element_type=f32)
        E = jnp.maximum(proj + bemb, 0.0)
        scores = jax.lax.dot_general(wattn, E, (((1,), (1,)), ((), ())),
                                     preferred_element_type=f32)
        idx_row = idx_ref[p:p + 1, :]
        OT = (iota_m == idx_row).astype(f32)
        s = jax.lax.dot_general(scores, OT, (((1,), (0,)), ((), ())),
                                preferred_element_type=f32)
        v = valid_ref[p:p + 1, :]
        s_m = jnp.where(v > 0.0, s, f32(-1e30))
        mx = jnp.max(s_m)
        e = jnp.exp(s_m - mx) * v
        attn = e / (jnp.sum(e) + f32(1e-9))
        w_m = jax.lax.dot_general(attn, OT, (((1,), (1,)), ((), ())),
                                  preferred_element_type=f32)
        pool = jax.lax.dot_general(w_m, E, (((1,), (0,)), ((), ())),
                                   preferred_element_type=f32)
        root = jax.lax.dot_general(OT[:, 1:2], E, (((0,), (0,)), ((), ())),
                                   preferred_element_type=f32)
        combs.append(jnp.concatenate([root, pool], axis=1))
    combined = jnp.concatenate(combs, axis=0)
    mlp = jnp.maximum(
        jax.lax.dot_general(combined, Wmlp_ref[...], (((1,), (0,)), ((), ())),
                            preferred_element_type=f32) + bmlp_ref[...], 0.0)
    mlp_ref[...] = mlp
    cost_ref[...] = jax.lax.dot_general(mlp, Wc_ref[...], (((1,), (0,)), ((), ())),
                                        preferred_element_type=f32) + bc_ref[...]
    card_ref[...] = jax.lax.dot_general(mlp, Wk_ref[...], (((1,), (0,)), ((), ())),
                                        preferred_element_type=f32) + bk_ref[...]


def kernel(trees, indexes, mask_padding, W_emb, b_emb, w_attn, W_mlp, b_mlp,
           W_cost, b_cost, W_card, b_card):
    f32 = jnp.float32
    idx = indexes.astype(jnp.int32)
    valid = jnp.logical_not(mask_padding).astype(f32)
    bemb2 = b_emb.reshape(1, _H).astype(f32)
    wattn2 = w_attn.reshape(1, _H).astype(f32)
    bmlp2 = b_mlp.reshape(1, _MO).astype(f32)
    bc2 = b_cost.reshape(1, 1).astype(f32)
    bk2 = b_card.reshape(1, 1).astype(f32)

    grid = (_B // _P,)
    rep = lambda *_: tuple(0 for _ in _)

    out = pl.pallas_call(
        _fused_kernel,
        grid=grid,
        in_specs=[
            pl.BlockSpec((_P, _F, _N), lambda i: (i, 0, 0)),
            pl.BlockSpec((_P, _N), lambda i: (i, 0)),
            pl.BlockSpec((_P, _N), lambda i: (i, 0)),
            pl.BlockSpec((_F, _H), lambda i: (0, 0)),
            pl.BlockSpec((1, _H), lambda i: (0, 0)),
            pl.BlockSpec((1, _H), lambda i: (0, 0)),
            pl.BlockSpec((2 * _H, _MO), lambda i: (0, 0)),
            pl.BlockSpec((1, _MO), lambda i: (0, 0)),
            pl.BlockSpec((_MO, 1), lambda i: (0, 0)),
            pl.BlockSpec((1, 1), lambda i: (0, 0)),
            pl.BlockSpec((_MO, 1), lambda i: (0, 0)),
            pl.BlockSpec((1, 1), lambda i: (0, 0)),
        ],
        out_specs=[
            pl.BlockSpec((_P, 1), lambda i: (i, 0)),
            pl.BlockSpec((_P, 1), lambda i: (i, 0)),
            pl.BlockSpec((_P, _MO), lambda i: (i, 0)),
        ],
        out_shape=[
            jax.ShapeDtypeStruct((_B, 1), f32),
            jax.ShapeDtypeStruct((_B, 1), f32),
            jax.ShapeDtypeStruct((_B, _MO), f32),
        ],
        compiler_params=pltpu.CompilerParams(
            dimension_semantics=("arbitrary",)),
    )(trees, idx, valid, W_emb, bemb2, wattn2, W_mlp, bmlp2,
      W_cost, bc2, W_card, bk2)
    pred_cost, pred_card, mlp_out = out
    return pred_cost, pred_card, mlp_out


# speedup vs baseline: 33.2719x; 4.7010x over previous
import jax
import jax.numpy as jnp
from jax.experimental import pallas as pl
from jax.experimental.pallas import tpu as pltpu

_B = 512
_N = 128
_F = 128
_H = 256
_MO = 64
_P = 16
_PN = _P * _N


def _fused_kernel(trees_ref, idxm_ref, rootidx_ref, Wemb_ref, bemb_ref,
                  Wcat_ref, bmlp_ref, Wck_ref, bck_ref,
                  cost_ref, card_ref, mlp_ref):
    f32 = jnp.float32
    bf16 = jnp.bfloat16
    i32 = jnp.int32

    Wemb = Wemb_ref[...].astype(bf16)
    A = jnp.transpose(trees_ref[...].astype(bf16), (0, 2, 1)).reshape(_PN, _F)
    proj = jax.lax.dot_general(A, Wemb, (((1,), (0,)), ((), ())),
                               preferred_element_type=f32)
    E = jnp.maximum(proj + bemb_ref[...], 0.0).astype(bf16)

    EW = jax.lax.dot_general(E, Wcat_ref[...], (((1,), (0,)), ((), ())),
                             preferred_element_type=f32)
    S_T = EW[:, 0:1].reshape(_P, _N)

    idxr = idxm_ref[0]
    iota_m = jax.lax.broadcasted_iota(i32, (_N, _PN), 0)
    OTV = (iota_m == idxr).astype(bf16)
    blk = (jax.lax.shift_right_logical(
        jax.lax.broadcasted_iota(i32, (_PN, _P), 0), 7)
        == jax.lax.broadcasted_iota(i32, (_PN, _P), 1)).astype(bf16)
    count = jax.lax.dot_general(OTV, blk, (((1,), (0,)), ((), ())),
                                preferred_element_type=f32)
    countT = jnp.transpose(count)

    mx = jnp.max(jnp.where(countT > 0.0, S_T, f32(-1e30)), axis=1,
                 keepdims=True)
    e = countT * jnp.exp(jnp.minimum(S_T - mx, 0.0))
    W_T = e / (jnp.sum(e, axis=1, keepdims=True) + f32(1e-9))

    lane_g = jax.lax.broadcasted_iota(i32, (_P, _PN), 1)
    row_p = jax.lax.broadcasted_iota(i32, (_P, _PN), 0)
    planmask = jax.lax.shift_right_logical(lane_g, 7) == row_p
    W_blk = jnp.where(planmask, W_T.reshape(1, _PN), 0.0)
    target = row_p * _N + rootidx_ref[0]
    R_sel = (lane_g == target).astype(f32)
    Sel = jnp.concatenate([W_blk, R_sel], axis=0).astype(bf16)
    RP = jax.lax.dot_general(Sel, EW[:, 1:129].astype(bf16),
                             (((1,), (0,)), ((), ())),
                             preferred_element_type=f32)

    mlp = jnp.maximum(RP[:_P, 0:_MO] + RP[_P:, _MO:2 * _MO]
                      + bmlp_ref[...], 0.0)
    mlp_ref[...] = mlp
    hk = jax.lax.dot_general(mlp, Wck_ref[...], (((1,), (0,)), ((), ())),
                             preferred_element_type=f32) + bck_ref[...]
    cost_ref[...] = hk[:, 0:1]
    card_ref[...] = hk[:, 1:2]


def kernel(trees, indexes, mask_padding, W_emb, b_emb, w_attn, W_mlp, b_mlp,
           W_cost, b_cost, W_card, b_card):
    f32 = jnp.float32
    idx = indexes.astype(jnp.int32)
    idx_masked = jnp.where(mask_padding, jnp.int32(-1), idx)
    idxm_flat = idx_masked.reshape(_B // _P, 1, _PN)
    rootidx = idx[:, 1].reshape(_B // _P, _P, 1)
    bemb2 = b_emb.reshape(1, _H).astype(f32)
    Wcat = jnp.concatenate(
        [w_attn.reshape(_H, 1), W_mlp[_H:], W_mlp[:_H]],
        axis=1).astype(jnp.bfloat16)
    bmlp2 = b_mlp.reshape(1, _MO).astype(f32)
    Wck = jnp.concatenate([W_cost, W_card], axis=1)
    bck = jnp.concatenate([b_cost, b_card]).reshape(1, 2).astype(f32)

    grid = (_B // _P,)

    out = pl.pallas_call(
        _fused_kernel,
        grid=grid,
        in_specs=[
            pl.BlockSpec((_P, _F, _N), lambda i: (i, 0, 0)),
            pl.BlockSpec((1, 1, _PN), lambda i: (i, 0, 0)),
            pl.BlockSpec((1, _P, 1), lambda i: (i, 0, 0)),
            pl.BlockSpec((_F, _H), lambda i: (0, 0)),
            pl.BlockSpec((1, _H), lambda i: (0, 0)),
            pl.BlockSpec((_H, 129), lambda i: (0, 0)),
            pl.BlockSpec((1, _MO), lambda i: (0, 0)),
            pl.BlockSpec((_MO, 2), lambda i: (0, 0)),
            pl.BlockSpec((1, 2), lambda i: (0, 0)),
        ],
        out_specs=[
            pl.BlockSpec((_P, 1), lambda i: (i, 0)),
            pl.BlockSpec((_P, 1), lambda i: (i, 0)),
            pl.BlockSpec((_P, _MO), lambda i: (i, 0)),
        ],
        out_shape=[
            jax.ShapeDtypeStruct((_B, 1), f32),
            jax.ShapeDtypeStruct((_B, 1), f32),
            jax.ShapeDtypeStruct((_B, _MO), f32),
        ],
        compiler_params=pltpu.CompilerParams(
            dimension_semantics=("arbitrary",)),
    )(trees, idxm_flat, rootidx, W_emb, bemb2, Wcat, bmlp2, Wck, bck)
    pred_cost, pred_card, mlp_out = out
    return pred_cost, pred_card, mlp_out


# speedup vs baseline: 39.1791x; 1.1775x over previous
import jax
import jax.numpy as jnp
from jax.experimental import pallas as pl
from jax.experimental.pallas import tpu as pltpu

_B = 512
_N = 128
_F = 128
_H = 256
_MO = 64
_P = 32
_PN = _P * _N


def _fused_kernel(trees_ref, idxm_ref, rootidx_ref, Wemb_ref, bemb_ref,
                  Wcat_ref, bmlp_ref, Wck_ref, bck_ref,
                  cost_ref, card_ref, mlp_ref):
    f32 = jnp.float32
    bf16 = jnp.bfloat16
    i32 = jnp.int32

    Wemb = Wemb_ref[...].astype(bf16)
    A = jnp.transpose(trees_ref[...].astype(bf16), (0, 2, 1)).reshape(_PN, _F)
    proj = jax.lax.dot_general(A, Wemb, (((1,), (0,)), ((), ())),
                               preferred_element_type=f32)
    E = jnp.maximum(proj + bemb_ref[...], 0.0).astype(bf16)

    EW = jax.lax.dot_general(E, Wcat_ref[...], (((1,), (0,)), ((), ())),
                             preferred_element_type=f32)
    S_T = EW[:, 0:1].reshape(_P, _N)

    idxr = idxm_ref[0]
    iota_m = jax.lax.broadcasted_iota(i32, (_N, _PN), 0)
    OTV = (iota_m == idxr).astype(bf16)
    blk = (jax.lax.shift_right_logical(
        jax.lax.broadcasted_iota(i32, (_PN, _P), 0), 7)
        == jax.lax.broadcasted_iota(i32, (_PN, _P), 1)).astype(bf16)
    count = jax.lax.dot_general(OTV, blk, (((1,), (0,)), ((), ())),
                                preferred_element_type=f32)
    countT = jnp.transpose(count)

    mx = jnp.max(jnp.where(countT > 0.0, S_T, f32(-1e30)), axis=1,
                 keepdims=True)
    e = countT * jnp.exp(jnp.minimum(S_T - mx, 0.0))
    W_T = e / (jnp.sum(e, axis=1, keepdims=True) + f32(1e-9))

    lane_g = jax.lax.broadcasted_iota(i32, (_P, _PN), 1)
    row_p = jax.lax.broadcasted_iota(i32, (_P, _PN), 0)
    planmask = jax.lax.shift_right_logical(lane_g, 7) == row_p
    W_blk = jnp.where(planmask, W_T.reshape(1, _PN), 0.0)
    target = row_p * _N + rootidx_ref[0]
    R_sel = (lane_g == target).astype(f32)
    Sel = jnp.concatenate([W_blk, R_sel], axis=0).astype(bf16)
    RP = jax.lax.dot_general(Sel, EW[:, 1:129].astype(bf16),
                             (((1,), (0,)), ((), ())),
                             preferred_element_type=f32)

    mlp = jnp.maximum(RP[:_P, 0:_MO] + RP[_P:, _MO:2 * _MO]
                      + bmlp_ref[...], 0.0)
    mlp_ref[...] = mlp
    hk = jax.lax.dot_general(mlp, Wck_ref[...], (((1,), (0,)), ((), ())),
                             preferred_element_type=f32) + bck_ref[...]
    cost_ref[...] = hk[:, 0:1]
    card_ref[...] = hk[:, 1:2]


def kernel(trees, indexes, mask_padding, W_emb, b_emb, w_attn, W_mlp, b_mlp,
           W_cost, b_cost, W_card, b_card):
    f32 = jnp.float32
    idx = indexes.astype(jnp.int32)
    idx_masked = jnp.where(mask_padding, jnp.int32(-1), idx)
    idxm_flat = idx_masked.reshape(_B // _P, 1, _PN)
    rootidx = idx[:, 1].reshape(_B // _P, _P, 1)
    bemb2 = b_emb.reshape(1, _H).astype(f32)
    Wcat = jnp.concatenate(
        [w_attn.reshape(_H, 1), W_mlp[_H:], W_mlp[:_H]],
        axis=1).astype(jnp.bfloat16)
    bmlp2 = b_mlp.reshape(1, _MO).astype(f32)
    Wck = jnp.concatenate([W_cost, W_card], axis=1)
    bck = jnp.concatenate([b_cost, b_card]).reshape(1, 2).astype(f32)

    grid = (_B // _P,)

    out = pl.pallas_call(
        _fused_kernel,
        grid=grid,
        in_specs=[
            pl.BlockSpec((_P, _F, _N), lambda i: (i, 0, 0)),
            pl.BlockSpec((1, 1, _PN), lambda i: (i, 0, 0)),
            pl.BlockSpec((1, _P, 1), lambda i: (i, 0, 0)),
            pl.BlockSpec((_F, _H), lambda i: (0, 0)),
            pl.BlockSpec((1, _H), lambda i: (0, 0)),
            pl.BlockSpec((_H, 129), lambda i: (0, 0)),
            pl.BlockSpec((1, _MO), lambda i: (0, 0)),
            pl.BlockSpec((_MO, 2), lambda i: (0, 0)),
            pl.BlockSpec((1, 2), lambda i: (0, 0)),
        ],
        out_specs=[
            pl.BlockSpec((_P, 1), lambda i: (i, 0)),
            pl.BlockSpec((_P, 1), lambda i: (i, 0)),
            pl.BlockSpec((_P, _MO), lambda i: (i, 0)),
        ],
        out_shape=[
            jax.ShapeDtypeStruct((_B, 1), f32),
            jax.ShapeDtypeStruct((_B, 1), f32),
            jax.ShapeDtypeStruct((_B, _MO), f32),
        ],
        compiler_params=pltpu.CompilerParams(
            dimension_semantics=("arbitrary",)),
    )(trees, idxm_flat, rootidx, W_emb, bemb2, Wcat, bmlp2, Wck, bck)
    pred_cost, pred_card, mlp_out = out
    return pred_cost, pred_card, mlp_out


# speedup vs baseline: 41.1448x; 1.0502x over previous
import jax
import jax.numpy as jnp
from jax.experimental import pallas as pl
from jax.experimental.pallas import tpu as pltpu

_B = 512
_N = 128
_F = 128
_H = 256
_MO = 64
_P = 64
_PN = _P * _N


def _fused_kernel(trees_ref, idxm_ref, rootidx_ref, Wemb_ref, bemb_ref,
                  Wcat_ref, bmlp_ref, Wck_ref, bck_ref,
                  cost_ref, card_ref, mlp_ref):
    f32 = jnp.float32
    bf16 = jnp.bfloat16
    i32 = jnp.int32

    Wemb = Wemb_ref[...].astype(bf16)
    A = jnp.transpose(trees_ref[...].astype(bf16), (0, 2, 1)).reshape(_PN, _F)
    proj = jax.lax.dot_general(A, Wemb, (((1,), (0,)), ((), ())),
                               preferred_element_type=f32)
    E = jnp.maximum(proj + bemb_ref[...], 0.0).astype(bf16)

    EW = jax.lax.dot_general(E, Wcat_ref[...], (((1,), (0,)), ((), ())),
                             preferred_element_type=f32)
    S_T = EW[:, 0:1].reshape(_P, _N)

    idxr = idxm_ref[0]
    iota_m = jax.lax.broadcasted_iota(i32, (_N, _PN), 0)
    OTV = (iota_m == idxr).astype(bf16)
    blk = (jax.lax.shift_right_logical(
        jax.lax.broadcasted_iota(i32, (_PN, _P), 0), 7)
        == jax.lax.broadcasted_iota(i32, (_PN, _P), 1)).astype(bf16)
    count = jax.lax.dot_general(OTV, blk, (((1,), (0,)), ((), ())),
                                preferred_element_type=f32)
    countT = jnp.transpose(count)

    mx = jnp.max(jnp.where(countT > 0.0, S_T, f32(-1e30)), axis=1,
                 keepdims=True)
    e = countT * jnp.exp(jnp.minimum(S_T - mx, 0.0))
    W_T = e / (jnp.sum(e, axis=1, keepdims=True) + f32(1e-9))

    lane_g = jax.lax.broadcasted_iota(i32, (_P, _PN), 1)
    row_p = jax.lax.broadcasted_iota(i32, (_P, _PN), 0)
    planmask = jax.lax.shift_right_logical(lane_g, 7) == row_p
    W_blk = jnp.where(planmask, W_T.reshape(1, _PN), 0.0)
    target = row_p * _N + rootidx_ref[0]
    R_sel = (lane_g == target).astype(f32)
    Sel = jnp.concatenate([W_blk, R_sel], axis=0).astype(bf16)
    RP = jax.lax.dot_general(Sel, EW[:, 1:129].astype(bf16),
                             (((1,), (0,)), ((), ())),
                             preferred_element_type=f32)

    mlp = jnp.maximum(RP[:_P, 0:_MO] + RP[_P:, _MO:2 * _MO]
                      + bmlp_ref[...], 0.0)
    mlp_ref[...] = mlp
    hk = jax.lax.dot_general(mlp, Wck_ref[...], (((1,), (0,)), ((), ())),
                             preferred_element_type=f32) + bck_ref[...]
    cost_ref[...] = hk[:, 0:1]
    card_ref[...] = hk[:, 1:2]


def kernel(trees, indexes, mask_padding, W_emb, b_emb, w_attn, W_mlp, b_mlp,
           W_cost, b_cost, W_card, b_card):
    f32 = jnp.float32
    idx = indexes.astype(jnp.int32)
    idx_masked = jnp.where(mask_padding, jnp.int32(-1), idx)
    idxm_flat = idx_masked.reshape(_B // _P, 1, _PN)
    rootidx = idx[:, 1].reshape(_B // _P, _P, 1)
    bemb2 = b_emb.reshape(1, _H).astype(f32)
    Wcat = jnp.concatenate(
        [w_attn.reshape(_H, 1), W_mlp[_H:], W_mlp[:_H]],
        axis=1).astype(jnp.bfloat16)
    bmlp2 = b_mlp.reshape(1, _MO).astype(f32)
    Wck = jnp.concatenate([W_cost, W_card], axis=1)
    bck = jnp.concatenate([b_cost, b_card]).reshape(1, 2).astype(f32)

    grid = (_B // _P,)

    out = pl.pallas_call(
        _fused_kernel,
        grid=grid,
        in_specs=[
            pl.BlockSpec((_P, _F, _N), lambda i: (i, 0, 0)),
            pl.BlockSpec((1, 1, _PN), lambda i: (i, 0, 0)),
            pl.BlockSpec((1, _P, 1), lambda i: (i, 0, 0)),
            pl.BlockSpec((_F, _H), lambda i: (0, 0)),
            pl.BlockSpec((1, _H), lambda i: (0, 0)),
            pl.BlockSpec((_H, 129), lambda i: (0, 0)),
            pl.BlockSpec((1, _MO), lambda i: (0, 0)),
            pl.BlockSpec((_MO, 2), lambda i: (0, 0)),
            pl.BlockSpec((1, 2), lambda i: (0, 0)),
        ],
        out_specs=[
            pl.BlockSpec((_P, 1), lambda i: (i, 0)),
            pl.BlockSpec((_P, 1), lambda i: (i, 0)),
            pl.BlockSpec((_P, _MO), lambda i: (i, 0)),
        ],
        out_shape=[
            jax.ShapeDtypeStruct((_B, 1), f32),
            jax.ShapeDtypeStruct((_B, 1), f32),
            jax.ShapeDtypeStruct((_B, _MO), f32),
        ],
        compiler_params=pltpu.CompilerParams(
            dimension_semantics=("arbitrary",)),
    )(trees, idxm_flat, rootidx, W_emb, bemb2, Wcat, bmlp2, Wck, bck)
    pred_cost, pred_card, mlp_out = out
    return pred_cost, pred_card, mlp_out
